# baseline (speedup 1.0000x reference)
import jax
import jax.numpy as jnp
from jax.experimental import pallas as pl
from jax.experimental.pallas import tpu as pltpu

D = 512
H = 16
K_TILE = 1024
N_K = (H * D) // K_TILE
ROW_TILE = 2048


def _combine_kernel(kp_ref, mem_ref, wout_ref, e_ref, acc_ref):
    i = pl.program_id(0)

    @pl.when(i == 0)
    def _():
        acc_ref[...] = jnp.zeros_like(acc_ref)

    acc_ref[...] += jax.lax.dot_general(
        kp_ref[...], mem_ref[...],
        (((0,), (0,)), ((), ())),
        preferred_element_type=jnp.float32,
    )

    @pl.when(i == N_K - 1)
    def _():
        e_ref[...] = jax.lax.dot_general(
            acc_ref[...] * (1.0 / H), wout_ref[...],
            (((1,), (1,)), ((), ())),
            preferred_element_type=jnp.float32,
        )


def _apply_kernel(z_ref, e_ref, out_ref):
    out_ref[...] = jnp.dot(
        z_ref[...], e_ref[...], preferred_element_type=jnp.float32
    )


@jax.jit
def kernel(z_query, key_proj_w, out_proj_w, memory):
    mem_2d = memory.reshape(H * D, D)
    e = pl.pallas_call(
        _combine_kernel,
        grid=(N_K,),
        in_specs=[
            pl.BlockSpec((K_TILE, D), lambda i: (i, 0)),
            pl.BlockSpec((K_TILE, D), lambda i: (i, 0)),
            pl.BlockSpec((D, D), lambda i: (0, 0)),
        ],
        out_specs=pl.BlockSpec((D, D), lambda i: (0, 0)),
        out_shape=jax.ShapeDtypeStruct((D, D), jnp.float32),
        scratch_shapes=[pltpu.VMEM((D, D), jnp.float32)],
        compiler_params=pltpu.CompilerParams(
            dimension_semantics=("arbitrary",),
        ),
    )(key_proj_w, mem_2d, out_proj_w)

    b = z_query.shape[0]
    out = pl.pallas_call(
        _apply_kernel,
        grid=(b // ROW_TILE,),
        in_specs=[
            pl.BlockSpec((ROW_TILE, D), lambda i: (i, 0)),
            pl.BlockSpec((D, D), lambda i: (0, 0)),
        ],
        out_specs=pl.BlockSpec((ROW_TILE, D), lambda i: (i, 0)),
        out_shape=jax.ShapeDtypeStruct((b, D), jnp.float32),
        compiler_params=pltpu.CompilerParams(
            dimension_semantics=("parallel",),
        ),
    )(z_query, e)
    return out
